# baseline (speedup 1.0000x reference)
import functools

import jax
import jax.numpy as jnp
from jax import lax
from jax.experimental import pallas as pl
from jax.experimental.pallas import tpu as pltpu
from jax.experimental.pallas import tpu_sc as plsc

BT = 256


def _router_body(hs_ref, gate_ref, wb1_ref, wb2_ref, pos1_ref, pos2_ref, bexp_ref,
                 nblk: int):
    x = hs_ref[...]
    T, E = x.shape[0], gate_ref.shape[1]
    logits = jnp.dot(x, gate_ref[...], preferred_element_type=jnp.float32)
    p = jax.nn.softmax(logits, axis=-1)
    idx = lax.broadcasted_iota(jnp.int32, p.shape, 1)
    m1 = jnp.max(p, axis=1, keepdims=True)
    i1 = jnp.min(jnp.where(p == m1, idx, E), axis=1, keepdims=True)
    sel1 = idx == i1
    p2 = jnp.where(sel1, -jnp.inf, p)
    m2 = jnp.max(p2, axis=1, keepdims=True)
    i2 = jnp.min(jnp.where(p2 == m2, idx, E), axis=1, keepdims=True)
    sel2 = idx == i2
    wsum = m1 + m2
    wb1_ref[...] = jnp.broadcast_to(m1 / wsum, wb1_ref.shape)
    wb2_ref[...] = jnp.broadcast_to(m2 / wsum, wb2_ref.shape)

    O1 = sel1.astype(jnp.float32)
    O2 = sel2.astype(jnp.float32)
    ri = lax.broadcasted_iota(jnp.int32, (T, T), 0)
    ci = lax.broadcasted_iota(jnp.int32, (T, T), 1)
    Lt = (ri > ci).astype(jnp.float32)
    P1 = jnp.dot(Lt, O1, preferred_element_type=jnp.float32)
    P2 = jnp.dot(Lt, O2, preferred_element_type=jnp.float32)
    cnt1 = jnp.sum(O1, axis=0, keepdims=True)
    cnt2 = jnp.sum(O2, axis=0, keepdims=True)
    cnt = cnt1 + cnt2
    nb = jnp.floor((cnt + (BT - 1)) / BT)
    ue_r = lax.broadcasted_iota(jnp.int32, (E, E), 0)
    ue_c = lax.broadcasted_iota(jnp.int32, (E, E), 1)
    U8 = (ue_r < ue_c).astype(jnp.float32)
    offb = jnp.dot(nb, U8, preferred_element_type=jnp.float32)
    off_pad = BT * offb
    pos1 = jnp.sum(sel1 * (off_pad + P1), axis=1, keepdims=True)
    pos2 = jnp.sum(sel2 * (off_pad + cnt1 + P2), axis=1, keepdims=True)
    pos1_ref[...] = pos1.astype(jnp.int32)
    pos2_ref[...] = pos2.astype(jnp.int32)

    ones_col = jnp.ones((T, 1), jnp.float32)
    cnt_col = lax.dot_general(O1 + O2, ones_col, (((0,), (0,)), ((), ())),
                              preferred_element_type=jnp.float32)
    nb_col = jnp.floor((cnt_col + (BT - 1)) / BT)
    L8 = (ue_r > ue_c).astype(jnp.float32)
    offb_col = jnp.dot(L8, nb_col, preferred_element_type=jnp.float32)
    biota = lax.broadcasted_iota(jnp.int32, (E, nblk), 1).astype(jnp.float32)
    in_e = (biota >= offb_col) & (biota < offb_col + nb_col)
    evals = lax.broadcasted_iota(jnp.int32, (E, nblk), 0).astype(jnp.float32)
    in_any = jnp.sum(in_e.astype(jnp.float32), axis=0, keepdims=True)
    bexp = jnp.sum(in_e * evals, axis=0, keepdims=True) + (E - 1) * (1.0 - in_any)
    bexp_ref[...] = bexp.astype(jnp.int32)


def _group_body(bexp_sref, xs_ref, wgu_ref, wdn_ref, ws_ref, y_ref):
    F = wdn_ref.shape[1]
    x = xs_ref[...]
    gu = jnp.dot(x, wgu_ref[0], preferred_element_type=jnp.float32)
    g = gu[:, :F]
    u = gu[:, F:]
    act = u * (g * jax.nn.sigmoid(g))
    y = jnp.dot(act, wdn_ref[0], preferred_element_type=jnp.float32)
    y_ref[...] = y * ws_ref[:, :1]


def _make_scatter(T, D, CAP, nw):
    ch = T // nw
    mesh = plsc.VectorSubcoreMesh(core_axis_name="c", subcore_axis_name="s")

    @functools.partial(
        pl.kernel,
        out_type=[jax.ShapeDtypeStruct((CAP, D), jnp.float32),
                  jax.ShapeDtypeStruct((CAP, 128), jnp.float32)],
        mesh=mesh,
        scratch_types=[
            pltpu.VMEM((ch,), jnp.int32),
            pltpu.VMEM((ch,), jnp.int32),
            pltpu.VMEM((ch, D), jnp.float32),
            pltpu.VMEM((ch, 128), jnp.float32),
            pltpu.SemaphoreType.DMA,
        ],
    )
    def scatter_k(hs_hbm, pos1_hbm, pos2_hbm, wb1_hbm, wb2_hbm,
                  xs_hbm, ws_hbm, idx1_v, idx2_v, rows_v, wrow_v, sem):
        wid = lax.axis_index("s") * 2 + lax.axis_index("c")
        base = wid * ch
        pltpu.sync_copy(pos1_hbm.at[pl.ds(base, ch)], idx1_v)
        pltpu.sync_copy(pos2_hbm.at[pl.ds(base, ch)], idx2_v)
        pltpu.sync_copy(hs_hbm.at[pl.ds(base, ch)], rows_v)
        pltpu.async_copy(rows_v, xs_hbm.at[idx1_v], sem).wait()
        pltpu.async_copy(rows_v, xs_hbm.at[idx2_v], sem).wait()
        pltpu.sync_copy(wb1_hbm.at[pl.ds(base, ch)], wrow_v)
        pltpu.async_copy(wrow_v, ws_hbm.at[idx1_v], sem).wait()
        pltpu.sync_copy(wb2_hbm.at[pl.ds(base, ch)], wrow_v)
        pltpu.async_copy(wrow_v, ws_hbm.at[idx2_v], sem).wait()

    return scatter_k


def _make_combine(T, D, CAP, nw):
    ch = 32
    npass = T // (nw * ch)
    mesh = plsc.VectorSubcoreMesh(core_axis_name="c", subcore_axis_name="s")

    @functools.partial(
        pl.kernel,
        out_type=jax.ShapeDtypeStruct((T, D), jnp.float32),
        mesh=mesh,
        scratch_types=[
            pltpu.VMEM((ch,), jnp.int32),
            pltpu.VMEM((ch,), jnp.int32),
            pltpu.VMEM((ch, D), jnp.float32),
            pltpu.VMEM((ch, D), jnp.float32),
            pltpu.SemaphoreType.DMA,
        ],
    )
    def combine_k(y_hbm, pos1_hbm, pos2_hbm, out_hbm,
                  idx1_v, idx2_v, buf1_v, buf2_v, sem):
        wid = lax.axis_index("s") * 2 + lax.axis_index("c")
        for it in range(npass):
            base = wid * (npass * ch) + it * ch
            pltpu.sync_copy(pos1_hbm.at[pl.ds(base, ch)], idx1_v)
            pltpu.sync_copy(pos2_hbm.at[pl.ds(base, ch)], idx2_v)
            pltpu.async_copy(y_hbm.at[idx1_v], buf1_v, sem).wait()
            pltpu.async_copy(y_hbm.at[idx2_v], buf2_v, sem).wait()

            def row_body(t, _):
                def lane_body(j, _):
                    sl = pl.ds(j * 16, 16)
                    buf1_v[t, sl] += buf2_v[t, sl]
                    return 0
                return lax.fori_loop(0, D // 16, lane_body, 0)

            lax.fori_loop(0, ch, row_body, 0)
            pltpu.sync_copy(buf1_v, out_hbm.at[pl.ds(base, ch)])

    return combine_k


def kernel(hidden_states, gate, gate_up_proj, down_proj):
    B, S, D = hidden_states.shape
    E, _, F2 = gate_up_proj.shape
    F = F2 // 2
    hs = hidden_states.reshape(-1, D)
    T = hs.shape[0]
    NBLK = 2 * T // BT + E
    CAP = NBLK * BT

    wb1, wb2, pos1, pos2, bexp = pl.pallas_call(
        functools.partial(_router_body, nblk=NBLK),
        grid=(1,),
        in_specs=[
            pl.BlockSpec((T, D), lambda i: (0, 0)),
            pl.BlockSpec((D, E), lambda i: (0, 0)),
        ],
        out_specs=[
            pl.BlockSpec((T, 128), lambda i: (0, 0)),
            pl.BlockSpec((T, 128), lambda i: (0, 0)),
            pl.BlockSpec((T, 1), lambda i: (0, 0)),
            pl.BlockSpec((T, 1), lambda i: (0, 0)),
            pl.BlockSpec((1, NBLK), lambda i: (0, 0)),
        ],
        out_shape=[
            jax.ShapeDtypeStruct((T, 128), jnp.float32),
            jax.ShapeDtypeStruct((T, 128), jnp.float32),
            jax.ShapeDtypeStruct((T, 1), jnp.int32),
            jax.ShapeDtypeStruct((T, 1), jnp.int32),
            jax.ShapeDtypeStruct((1, NBLK), jnp.int32),
        ],
    )(hs, gate)

    pos1f = pos1.reshape(T)
    pos2f = pos2.reshape(T)
    nw = 32
    xs, ws = _make_scatter(T, D, CAP, nw)(hs, pos1f, pos2f, wb1, wb2)

    y = pl.pallas_call(
        _group_body,
        grid_spec=pltpu.PrefetchScalarGridSpec(
            num_scalar_prefetch=1,
            grid=(NBLK,),
            in_specs=[
                pl.BlockSpec((BT, D), lambda b, be: (b, 0)),
                pl.BlockSpec((1, D, F2), lambda b, be: (be[b], 0, 0)),
                pl.BlockSpec((1, F, D), lambda b, be: (be[b], 0, 0)),
                pl.BlockSpec((BT, 128), lambda b, be: (b, 0)),
            ],
            out_specs=pl.BlockSpec((BT, D), lambda b, be: (b, 0)),
        ),
        out_shape=jax.ShapeDtypeStruct((CAP, D), jnp.float32),
        compiler_params=pltpu.CompilerParams(
            dimension_semantics=("arbitrary",),
        ),
    )(bexp.reshape(NBLK), xs, gate_up_proj, down_proj, ws)

    out = _make_combine(T, D, CAP, nw)(y, pos1f, pos2f)
    return out.reshape(B, S, D)


# speedup vs baseline: 1.4697x; 1.4697x over previous
import jax
import jax.numpy as jnp
from jax.experimental import pallas as pl
from jax.experimental.pallas import tpu as pltpu


def _moe_body(hs_ref, gate_ref, gu_ref, dn_ref, out_ref, w_ref):
    e = pl.program_id(0)
    T, E = w_ref.shape
    F = dn_ref.shape[1]

    @pl.when(e == 0)
    def _router():
        x = hs_ref[...]
        logits = jnp.dot(x, gate_ref[...], preferred_element_type=jnp.float32)
        p = jax.nn.softmax(logits, axis=-1)
        idx = jax.lax.broadcasted_iota(jnp.int32, p.shape, 1)
        m1 = jnp.max(p, axis=1, keepdims=True)
        i1 = jnp.min(jnp.where(p == m1, idx, E), axis=1, keepdims=True)
        sel1 = idx == i1
        p2 = jnp.where(sel1, -jnp.inf, p)
        m2 = jnp.max(p2, axis=1, keepdims=True)
        i2 = jnp.min(jnp.where(p2 == m2, idx, E), axis=1, keepdims=True)
        sel2 = idx == i2
        wsum = m1 + m2
        w = jnp.where(sel1, m1, jnp.where(sel2, m2, 0.0)) / wsum
        w_ref[...] = w
        out_ref[...] = jnp.zeros_like(out_ref)

    x = hs_ref[...].astype(jnp.bfloat16)
    wgu = gu_ref[0].astype(jnp.bfloat16)
    gu = jnp.dot(x, wgu, preferred_element_type=jnp.float32)
    g = gu[:, :F]
    u = gu[:, F:]
    act = (u * (g * jax.nn.sigmoid(g))).astype(jnp.bfloat16)
    wdn = dn_ref[0].astype(jnp.bfloat16)
    d = jnp.dot(act, wdn, preferred_element_type=jnp.float32)
    lane = jax.lax.broadcasted_iota(jnp.int32, (T, E), 1)
    w_col = jnp.sum(jnp.where(lane == e, w_ref[...], 0.0), axis=1, keepdims=True)
    out_ref[...] += w_col * d


def kernel(hidden_states, gate, gate_up_proj, down_proj):
    B, S, D = hidden_states.shape
    E, _, F2 = gate_up_proj.shape
    F = F2 // 2
    hs = hidden_states.reshape(-1, D)
    T = hs.shape[0]

    out = pl.pallas_call(
        _moe_body,
        grid=(E,),
        in_specs=[
            pl.BlockSpec((T, D), lambda e: (0, 0)),
            pl.BlockSpec((D, E), lambda e: (0, 0)),
            pl.BlockSpec((1, D, F2), lambda e: (e, 0, 0)),
            pl.BlockSpec((1, F, D), lambda e: (e, 0, 0)),
        ],
        out_specs=pl.BlockSpec((T, D), lambda e: (0, 0)),
        out_shape=jax.ShapeDtypeStruct((T, D), jnp.float32),
        scratch_shapes=[pltpu.VMEM((T, E), jnp.float32)],
        compiler_params=pltpu.CompilerParams(
            dimension_semantics=("arbitrary",),
        ),
    )(hs, gate, gate_up_proj, down_proj)
    return out.reshape(B, S, D)
